# baseline (speedup 1.0000x reference)
import functools

import jax
import jax.numpy as jnp
from jax import lax
from jax.experimental import pallas as pl
from jax.experimental.pallas import tpu as pltpu
from jax.experimental.pallas import tpu_sc as plsc

TAU_CONVERGE = 8.0
_NC, _NS = 2, 16


def _index_kernel(zp_ref, zc_ref, cb_ref, csq_ref, mc_ref, g_ref):
    zc = zc_ref[0]
    zp = zp_ref[0]
    cb = cb_ref[0]
    c_sq = csq_ref[0]
    mcols = mc_ref[...]
    M = cb.shape[0]
    TL = zc.shape[1]
    h = pl.program_id(0)
    pid = pl.program_id(1)
    diff = zc - zp
    vel = jnp.sqrt(jnp.sum(diff * diff, axis=0, keepdims=True))
    converged = vel < TAU_CONVERGE
    dots = jax.lax.dot_general(cb, zc, (((1,), (0,)), ((), ())),
                               preferred_element_type=jnp.float32)
    dists = c_sq - 2.0 * dots
    mn = jnp.min(dists, axis=0, keepdims=True)
    onehot = (dists == mn).astype(jnp.float32)
    mi = jax.lax.dot_general(mcols, onehot, (((0,), (0,)), ((), ())),
                             preferred_element_type=jnp.float32)
    idx = jnp.round((256.0 * mi[0:1] + mi[1:2]) / mi[2:3]).astype(jnp.int32)
    H = pl.num_programs(0)
    N = pl.num_programs(1) * TL
    lanes = jax.lax.broadcasted_iota(jnp.int32, (1, TL), 1)
    pair = (pid * TL + lanes) * H + h
    snap = N * H + h * M + idx
    g_ref[0] = jnp.where(converged, snap, pair)


def _tc_indices(z_prev, z_current, codebook, TL):
    B, L, dim = z_current.shape
    H, M, d = codebook.shape
    N = B * L
    zp = z_prev.reshape(N, H, d).transpose(1, 2, 0)
    zc = z_current.reshape(N, H, d).transpose(1, 2, 0)
    c_sq = jnp.sum(codebook * codebook, axis=-1, keepdims=True)
    marange = jnp.arange(M, dtype=jnp.int32)
    mcols = jnp.stack(
        [(marange // 256).astype(jnp.float32),
         (marange % 256).astype(jnp.float32),
         jnp.ones((M,), jnp.float32)], axis=1)
    return pl.pallas_call(
        _index_kernel,
        grid=(H, N // TL),
        in_specs=[
            pl.BlockSpec((1, d, TL), lambda h, i: (h, 0, i)),
            pl.BlockSpec((1, d, TL), lambda h, i: (h, 0, i)),
            pl.BlockSpec((1, M, d), lambda h, i: (h, 0, 0)),
            pl.BlockSpec((1, M, 1), lambda h, i: (h, 0, 0)),
            pl.BlockSpec((M, 3), lambda h, i: (0, 0)),
        ],
        out_specs=pl.BlockSpec((1, 1, TL), lambda h, i: (h, 0, i)),
        out_shape=jax.ShapeDtypeStruct((H, 1, N), jnp.int32),
    )(zp, zc, codebook, c_sq, mcols)


def _make_sc_gather(d, n_pairs):
    NW = _NC * _NS
    b_per_w = n_pairs // NW
    CH = 2048
    RPD = 128
    n_ch = b_per_w // CH
    mesh = plsc.VectorSubcoreMesh(core_axis_name="c", subcore_axis_name="s")

    @functools.partial(
        pl.kernel, mesh=mesh,
        compiler_params=pltpu.CompilerParams(use_tc_tiling_on_sc=False),
        out_type=jax.ShapeDtypeStruct((n_pairs, d), jnp.float32),
        scratch_types=[
            pltpu.VMEM((CH // RPD, RPD), jnp.int32),
            pltpu.VMEM((CH, d), jnp.float32),
            pltpu.SemaphoreType.DMA,
        ],
    )
    def sc_gather(table_hbm, idx2_hbm, out_hbm, idx_v, rows_v, sem):
        wid = lax.axis_index("s") * _NC + lax.axis_index("c")
        base = pl.multiple_of(wid * b_per_w, b_per_w)
        for j in range(n_ch):
            off = pl.multiple_of(base + j * CH, CH)
            pltpu.sync_copy(
                idx2_hbm.at[pl.ds(pl.multiple_of(off // RPD, CH // RPD),
                                  CH // RPD)], idx_v)
            cps = [pltpu.async_copy(table_hbm.at[idx_v.at[r]],
                                    rows_v.at[pl.ds(r * RPD, RPD)], sem)
                   for r in range(CH // RPD)]
            for cp in cps:
                cp.wait()
            pltpu.sync_copy(rows_v, out_hbm.at[pl.ds(pl.multiple_of(off, CH),
                                                     CH)])

    return sc_gather


@jax.jit
def kernel(z_prev, z_current, codebook):
    B, L, dim = z_current.shape
    H, M, d = codebook.shape
    N = B * L
    n_pairs = N * H
    g = _tc_indices(z_prev, z_current, codebook, TL=1024)
    gp = g.reshape(H, N).transpose(1, 0).reshape(n_pairs)
    idx2 = gp.reshape(n_pairs // 128, 128)
    table = jnp.concatenate(
        [z_current.reshape(n_pairs, d), codebook.reshape(H * M, d)], axis=0)
    out = _make_sc_gather(d, n_pairs)(table, idx2)
    return out.reshape(B, L, dim)


# speedup vs baseline: 1.5096x; 1.5096x over previous
import jax
import jax.numpy as jnp
from jax.experimental import pallas as pl

TAU_CONVERGE = 8.0


def _snap_kernel(zp_ref, zc_ref, cb_ref, csq_ref, out_ref):
    zc = zc_ref[0]
    zp = zp_ref[0]
    cb = cb_ref[0]
    c_sq = csq_ref[0]
    diff = zc - zp
    vel = jnp.sqrt(jnp.sum(diff * diff, axis=0, keepdims=True))
    converged = vel < TAU_CONVERGE
    dots = jax.lax.dot_general(cb, zc, (((1,), (0,)), ((), ())),
                               preferred_element_type=jnp.float32)
    dists = c_sq - 2.0 * dots
    mn = jnp.min(dists, axis=0, keepdims=True)
    onehot = (dists == mn).astype(jnp.float32)
    cnt = jnp.sum(onehot, axis=0, keepdims=True)
    entries = jax.lax.dot_general(cb, onehot, (((0,), (0,)), ((), ())),
                                  preferred_element_type=jnp.float32)
    entries = entries / cnt
    out_ref[0] = jnp.where(converged, entries, zc)


@jax.jit
def kernel(z_prev, z_current, codebook):
    B, L, dim = z_current.shape
    H, M, d = codebook.shape
    N = B * L
    TL = 1024
    zp = z_prev.reshape(N, H, d).transpose(1, 2, 0)
    zc = z_current.reshape(N, H, d).transpose(1, 2, 0)
    c_sq = jnp.sum(codebook * codebook, axis=-1, keepdims=True)
    out = pl.pallas_call(
        _snap_kernel,
        grid=(H, N // TL),
        in_specs=[
            pl.BlockSpec((1, d, TL), lambda h, i: (h, 0, i)),
            pl.BlockSpec((1, d, TL), lambda h, i: (h, 0, i)),
            pl.BlockSpec((1, M, d), lambda h, i: (h, 0, 0)),
            pl.BlockSpec((1, M, 1), lambda h, i: (h, 0, 0)),
        ],
        out_specs=pl.BlockSpec((1, d, TL), lambda h, i: (h, 0, i)),
        out_shape=jax.ShapeDtypeStruct((H, d, N), jnp.float32),
    )(zp, zc, codebook, c_sq)
    return out.transpose(2, 0, 1).reshape(B, L, dim)


# speedup vs baseline: 1.7721x; 1.1739x over previous
import jax
import jax.numpy as jnp
from jax.experimental import pallas as pl

TAU_CONVERGE = 8.0


def _snap_kernel(zp_ref, zc_ref, cb_ref, csq_ref, out_ref):
    zc = zc_ref[0]
    zp = zp_ref[0]
    cb = cb_ref[0]
    c_sq = csq_ref[0]
    diff = zc - zp
    vel = jnp.sqrt(jnp.sum(diff * diff, axis=0, keepdims=True))
    converged = vel < TAU_CONVERGE
    dots = jax.lax.dot_general(cb, zc, (((1,), (0,)), ((), ())),
                               preferred_element_type=jnp.float32)
    dists = c_sq - 2.0 * dots
    mn = jnp.min(dists, axis=0, keepdims=True)
    onehot = (dists == mn).astype(jnp.float32)
    cnt = jnp.sum(onehot, axis=0, keepdims=True)
    entries = jax.lax.dot_general(cb, onehot, (((0,), (0,)), ((), ())),
                                  preferred_element_type=jnp.float32)
    entries = entries / cnt
    out_ref[0] = jnp.where(converged, entries, zc)


@jax.jit
def kernel(z_prev, z_current, codebook):
    B, L, dim = z_current.shape
    H, M, d = codebook.shape
    N = B * L
    TL = 2048
    zp = z_prev.reshape(N, H, d).transpose(1, 2, 0)
    zc = z_current.reshape(N, H, d).transpose(1, 2, 0)
    c_sq = jnp.sum(codebook * codebook, axis=-1, keepdims=True)
    out = pl.pallas_call(
        _snap_kernel,
        grid=(H, N // TL),
        in_specs=[
            pl.BlockSpec((1, d, TL), lambda h, i: (h, 0, i)),
            pl.BlockSpec((1, d, TL), lambda h, i: (h, 0, i)),
            pl.BlockSpec((1, M, d), lambda h, i: (h, 0, 0)),
            pl.BlockSpec((1, M, 1), lambda h, i: (h, 0, 0)),
        ],
        out_specs=pl.BlockSpec((1, d, TL), lambda h, i: (h, 0, i)),
        out_shape=jax.ShapeDtypeStruct((H, d, N), jnp.float32),
    )(zp, zc, codebook, c_sq)
    return out.transpose(2, 0, 1).reshape(B, L, dim)
